# baseline (speedup 1.0000x reference)
import functools

import jax
import jax.numpy as jnp
from jax.experimental import pallas as pl

K0 = 10
KT = 110

QB = 256
CK = 2048
KPAD = 100352


def _score_kernel(q_ref, k_ref, s_ref):
    q = q_ref[...]
    k = k_ref[...]
    s_ref[...] = jax.lax.dot_general(
        q, k, (((1,), (1,)), ((), ())), preferred_element_type=jnp.float32)


def kernel(queries, keys):
    nq, d = queries.shape
    nk, _ = keys.shape
    keys_p = jnp.pad(keys, ((0, KPAD - nk), (0, 0)))
    grid = (nq // QB, KPAD // CK)
    scores = pl.pallas_call(
        _score_kernel,
        grid=grid,
        in_specs=[
            pl.BlockSpec((QB, d), lambda i, j: (i, 0)),
            pl.BlockSpec((CK, d), lambda i, j: (j, 0)),
        ],
        out_specs=pl.BlockSpec((QB, CK), lambda i, j: (i, j)),
        out_shape=jax.ShapeDtypeStruct((nq, KPAD), jnp.float32),
    )(queries, keys_p)
    vals, idx = jax.lax.top_k(scores[:, :nk], KT)
    return vals[:, K0:KT], idx[:, K0:KT]
